# baseline (speedup 1.0000x reference)
import functools

import jax
import jax.numpy as jnp
from jax import lax
from jax.experimental import pallas as pl
from jax.experimental.pallas import tpu as pltpu
from jax.experimental.pallas import tpu_sc as plsc

N = 10000
E = 320000
D = 128
D_OUT = 64
DK = 16
NC, NS = 2, 16
NW = NC * NS
KE = 125
ROWS_E = E // KE
RPW = ROWS_E // NW
EPT = E // NW
N_PAD = 10240
NPT = N_PAD // NS
ZB = 16
NH = 5120
TABR = NH * DK // 128
DGR = 2 * TABR
RB = 2000


def _sc_deg_body(dst_hbm, out0_hbm, out1_hbm, idx_v, tab_v, deg_sh):
    cid = lax.axis_index("c")
    sid = lax.axis_index("s")
    wid = cid * NS + sid
    zero16 = jnp.zeros((16,), jnp.float32)
    ones16 = jnp.ones((16,), jnp.float32)
    lanes = lax.iota(jnp.int32, 16)

    def zero_tab():
        def zbody(i, c):
            for u in range(8):
                tab_v[i, pl.ds(u * 16, 16)] = zero16
            return c
        lax.fori_loop(0, TABR, zbody, 0)

    zero_tab()
    pltpu.sync_copy(tab_v.at[pl.ds(0, DGR // NS)],
                    deg_sh.at[pl.ds(sid * (DGR // NS), DGR // NS)])
    pltpu.sync_copy(dst_hbm.at[pl.ds(wid * EPT, EPT)], idx_v)
    plsc.subcore_barrier()

    for r in range(2):
        lo = r * NH

        def sbody(g, c):
            d = idx_v[pl.ds(g * 16, 16)]
            in_r = (d >= lo) & (d < lo + NH)
            a = (d - lo) * 16 + lanes
            a = jnp.where(in_r, a, 0)
            plsc.addupdate_scatter(
                tab_v, [lax.shift_right_logical(a, 7), a & 127], ones16,
                mask=in_r)
            return c

        lax.fori_loop(0, EPT // 16, sbody, 0)

        def mbody(g, c):
            idx16 = lanes + (r * TABR + g * 16)
            pltpu.sync_copy(tab_v.at[pl.ds(g * 16, 16)],
                            deg_sh.at[idx16], add=True)
            return c

        lax.fori_loop(0, TABR // 16, mbody, 0)
        if r == 0:
            zero_tab()
    plsc.subcore_barrier()

    @pl.when(cid == 0)
    def _():
        pltpu.sync_copy(deg_sh.at[pl.ds(sid * (DGR // NS), DGR // NS)],
                        out0_hbm.at[pl.ds(sid * (DGR // NS), DGR // NS)])

    @pl.when(cid == 1)
    def _():
        pltpu.sync_copy(deg_sh.at[pl.ds(sid * (DGR // NS), DGR // NS)],
                        out1_hbm.at[pl.ds(sid * (DGR // NS), DGR // NS)])


def _sc_agg_body(m_hbm, src2_hbm, dst2_hbm, out0_hbm, out1_hbm,
                 sidx_v, didx_v, rows0_v, rows1_v, zb_v, sem0, sem1, acc_sh):
    cid = lax.axis_index("c")
    sid = lax.axis_index("s")
    wid = cid * NS + sid
    zero16 = jnp.zeros((16,), jnp.float32)
    for i in range(ZB):
        for t in range(D // 16):
            zb_v[i, pl.ds(t * 16, 16)] = zero16
    base = sid * NPT
    for k in range(NPT // ZB):
        pltpu.sync_copy(zb_v, acc_sh.at[pl.ds(base + k * ZB, ZB)])
    pltpu.sync_copy(src2_hbm.at[pl.ds(wid * RPW, RPW)], sidx_v)
    plsc.subcore_barrier()

    bufs = (rows0_v, rows1_v)
    sems = (sem0, sem1)
    pltpu.async_copy(m_hbm.at[sidx_v.at[0]], rows0_v, sem0)
    pltpu.async_copy(m_hbm.at[sidx_v.at[1]], rows1_v, sem1)

    def group(g, c):
        pltpu.sync_copy(dst2_hbm.at[pl.ds(wid * RPW + g * 8, 8)], didx_v)
        for u in range(8):
            b = u % 2
            j = g * 8 + u
            pltpu.make_async_copy(
                m_hbm.at[sidx_v.at[j]], bufs[b], sems[b]).wait()
            pltpu.sync_copy(bufs[b], acc_sh.at[didx_v.at[u]], add=True)

            @pl.when(j + 2 < RPW)
            def _():
                pltpu.async_copy(m_hbm.at[sidx_v.at[j + 2]], bufs[b], sems[b])
        return c

    lax.fori_loop(0, RPW // 8, group, 0)
    plsc.subcore_barrier()

    @pl.when(cid == 0)
    def _():
        pltpu.sync_copy(acc_sh.at[pl.ds(base, NPT)],
                        out0_hbm.at[pl.ds(base, NPT)])

    @pl.when(cid == 1)
    def _():
        pltpu.sync_copy(acc_sh.at[pl.ds(base, NPT)],
                        out1_hbm.at[pl.ds(base, NPT)])


@functools.cache
def _sc_kernels():
    mesh = plsc.VectorSubcoreMesh(
        core_axis_name="c", subcore_axis_name="s",
        num_cores=NC, num_subcores=NS)
    sc_deg = pl.kernel(
        _sc_deg_body,
        out_type=(jax.ShapeDtypeStruct((DGR, 128), jnp.float32),
                  jax.ShapeDtypeStruct((DGR, 128), jnp.float32)),
        mesh=mesh,
        scratch_types=[
            pltpu.VMEM((EPT,), jnp.int32),
            pltpu.VMEM((TABR, 128), jnp.float32),
            pltpu.VMEM_SHARED((DGR, 128), jnp.float32),
        ],
        compiler_params=pltpu.CompilerParams(needs_layout_passes=False),
    )
    sc_agg = pl.kernel(
        _sc_agg_body,
        out_type=(jax.ShapeDtypeStruct((N_PAD, D), jnp.float32),
                  jax.ShapeDtypeStruct((N_PAD, D), jnp.float32)),
        mesh=mesh,
        scratch_types=[
            pltpu.VMEM((RPW, KE), jnp.int32),
            pltpu.VMEM((8, KE), jnp.int32),
            pltpu.VMEM((KE, D), jnp.float32),
            pltpu.VMEM((KE, D), jnp.float32),
            pltpu.VMEM((ZB, D), jnp.float32),
            pltpu.SemaphoreType.DMA,
            pltpu.SemaphoreType.DMA,
            pltpu.VMEM_SHARED((N_PAD, D), jnp.float32),
        ],
    )
    return sc_deg, sc_agg


def _tc_pre_body(d0_ref, d1_ref, x_ref, w1_ref, m1_ref, dinv_ref):
    deg = jnp.sum(d0_ref[...] + d1_ref[...], axis=1, keepdims=True) + 1.0
    dinv = lax.rsqrt(deg)
    t = jnp.dot(x_ref[...], w1_ref[...], preferred_element_type=jnp.float32)
    m1_ref[...] = dinv * t
    dinv_ref[...] = jnp.broadcast_to(dinv, (RB, DK))


def _tc_mid_body(a0_ref, a1_ref, m1_ref, dinv_ref, b1_ref, m2_ref):
    dv = dinv_ref[:, 0:1]
    h = jnp.maximum(
        dv * (a0_ref[...] + a1_ref[...] + m1_ref[...]) + b1_ref[...], 0.0)
    m2_ref[...] = dv * h


def _tc_out_body(a0_ref, a1_ref, m2_ref, dinv_ref, w2_ref, b2_ref,
                 w3_ref, b3_ref, mu_ref, lv_ref):
    dv = dinv_ref[:, 0:1]
    g = dv * (a0_ref[...] + a1_ref[...] + m2_ref[...])
    mu_ref[...] = jnp.dot(
        g, w2_ref[...], preferred_element_type=jnp.float32) + b2_ref[...]
    lv_ref[...] = jnp.dot(
        g, w3_ref[...], preferred_element_type=jnp.float32) + b3_ref[...]


def _row_spec(w):
    return pl.BlockSpec((RB, w), lambda i: (i, 0))


def _full_spec(h, w):
    return pl.BlockSpec((h, w), lambda i: (0, 0))


_GRID = (N_PAD // RB,)

_tc_pre = pl.pallas_call(
    _tc_pre_body,
    grid=_GRID,
    in_specs=[_row_spec(DK), _row_spec(DK), _row_spec(D), _full_spec(D, D)],
    out_specs=[_row_spec(D), _row_spec(DK)],
    out_shape=[jax.ShapeDtypeStruct((N, D), jnp.float32),
               jax.ShapeDtypeStruct((N, DK), jnp.float32)],
)

_tc_mid = pl.pallas_call(
    _tc_mid_body,
    grid=_GRID,
    in_specs=[_row_spec(D), _row_spec(D), _row_spec(D), _row_spec(DK),
              _full_spec(1, D)],
    out_specs=_row_spec(D),
    out_shape=jax.ShapeDtypeStruct((N, D), jnp.float32),
)

_tc_out = pl.pallas_call(
    _tc_out_body,
    grid=_GRID,
    in_specs=[_row_spec(D), _row_spec(D), _row_spec(D), _row_spec(DK),
              _full_spec(D, D_OUT), _full_spec(1, D_OUT),
              _full_spec(D, D_OUT), _full_spec(1, D_OUT)],
    out_specs=[_row_spec(D_OUT), _row_spec(D_OUT)],
    out_shape=[jax.ShapeDtypeStruct((N, D_OUT), jnp.float32),
               jax.ShapeDtypeStruct((N, D_OUT), jnp.float32)],
)


def kernel(x, edge_index, W1, b1, W2, b2, W3, b3):
    sc_deg, sc_agg = _sc_kernels()
    src2 = edge_index[0].reshape(ROWS_E, KE)
    dst2 = edge_index[1].reshape(ROWS_E, KE)
    deg0, deg1 = sc_deg(edge_index[1])
    m1, dinv16 = _tc_pre(deg0.reshape(N_PAD, DK), deg1.reshape(N_PAD, DK),
                         x, W1)
    a0, a1 = sc_agg(m1, src2, dst2)
    m2 = _tc_mid(a0, a1, m1, dinv16, b1.reshape(1, D))
    b0, b1agg = sc_agg(m2, src2, dst2)
    mu, lv = _tc_out(b0, b1agg, m2, dinv16, W2, b2.reshape(1, D_OUT),
                     W3, b3.reshape(1, D_OUT))
    return mu, lv


# speedup vs baseline: 1.0102x; 1.0102x over previous
import functools

import jax
import jax.numpy as jnp
from jax import lax
from jax.experimental import pallas as pl
from jax.experimental.pallas import tpu as pltpu
from jax.experimental.pallas import tpu_sc as plsc

N = 10000
E = 320000
D = 128
D_OUT = 64
DK = 16
NC, NS = 2, 16
NW = NC * NS
KE = 125
ROWS_E = E // KE
RPW = ROWS_E // NW
EPT = E // NW
N_PAD = 10240
NPT = N_PAD // NS
ZB = 16
NH = 5120
TABR = NH * DK // 128
DGR = 2 * TABR
RB = 2000


def _sc_deg_body(dst_hbm, out0_hbm, out1_hbm, idx_v, tab_v, msem, deg_sh):
    cid = lax.axis_index("c")
    sid = lax.axis_index("s")
    wid = cid * NS + sid
    zero16 = jnp.zeros((16,), jnp.float32)
    ones16 = jnp.ones((16,), jnp.float32)
    lanes = lax.iota(jnp.int32, 16)

    def zero_tab():
        def zbody(i, c):
            for u in range(8):
                tab_v[i, pl.ds(u * 16, 16)] = zero16
            return c
        lax.fori_loop(0, TABR, zbody, 0)

    zero_tab()
    pltpu.sync_copy(tab_v.at[pl.ds(0, DGR // NS)],
                    deg_sh.at[pl.ds(sid * (DGR // NS), DGR // NS)])
    pltpu.sync_copy(dst_hbm.at[pl.ds(wid * EPT, EPT)], idx_v)
    plsc.subcore_barrier()

    for r in range(2):
        lo = r * NH

        def sbody(g, c):
            d = idx_v[pl.ds(g * 16, 16)]
            in_r = (d >= lo) & (d < lo + NH)
            a = (d - lo) * 16 + lanes
            a = jnp.where(in_r, a, 0)
            plsc.addupdate_scatter(
                tab_v, [lax.shift_right_logical(a, 7), a & 127], ones16,
                mask=in_r)
            return c

        lax.fori_loop(0, EPT // 16, sbody, 0)

        def mbody(g, c):
            idx16 = lanes + (r * TABR + g * 16)
            pltpu.async_copy(tab_v.at[pl.ds(g * 16, 16)],
                             deg_sh.at[idx16], msem, add=True)
            return c

        lax.fori_loop(0, TABR // 16, mbody, 0)

        def dbody(g, c):
            pltpu.make_async_copy(tab_v.at[pl.ds(g * 16, 16)],
                                  deg_sh.at[lanes], msem).wait()
            return c

        lax.fori_loop(0, TABR // 16, dbody, 0)
        if r == 0:
            zero_tab()
    plsc.subcore_barrier()

    @pl.when(cid == 0)
    def _():
        pltpu.sync_copy(deg_sh.at[pl.ds(sid * (DGR // NS), DGR // NS)],
                        out0_hbm.at[pl.ds(sid * (DGR // NS), DGR // NS)])

    @pl.when(cid == 1)
    def _():
        pltpu.sync_copy(deg_sh.at[pl.ds(sid * (DGR // NS), DGR // NS)],
                        out1_hbm.at[pl.ds(sid * (DGR // NS), DGR // NS)])


def _sc_agg_body(m_hbm, src2_hbm, dst2_hbm, out0_hbm, out1_hbm,
                 sidx_v, didx_v, rows0_v, rows1_v, zb_v, sem0, sem1, acc_sh):
    cid = lax.axis_index("c")
    sid = lax.axis_index("s")
    wid = cid * NS + sid
    zero16 = jnp.zeros((16,), jnp.float32)
    for i in range(ZB):
        for t in range(D // 16):
            zb_v[i, pl.ds(t * 16, 16)] = zero16
    base = sid * NPT
    for k in range(NPT // ZB):
        pltpu.sync_copy(zb_v, acc_sh.at[pl.ds(base + k * ZB, ZB)])
    pltpu.sync_copy(src2_hbm.at[pl.ds(wid * RPW, RPW)], sidx_v)
    plsc.subcore_barrier()

    bufs = (rows0_v, rows1_v)
    sems = (sem0, sem1)
    pltpu.async_copy(m_hbm.at[sidx_v.at[0]], rows0_v, sem0)
    pltpu.async_copy(m_hbm.at[sidx_v.at[1]], rows1_v, sem1)

    def group(g, c):
        pltpu.sync_copy(dst2_hbm.at[pl.ds(wid * RPW + g * 8, 8)], didx_v)
        for u in range(8):
            b = u % 2
            j = g * 8 + u
            pltpu.make_async_copy(
                m_hbm.at[sidx_v.at[j]], bufs[b], sems[b]).wait()
            pltpu.sync_copy(bufs[b], acc_sh.at[didx_v.at[u]], add=True)

            @pl.when(j + 2 < RPW)
            def _():
                pltpu.async_copy(m_hbm.at[sidx_v.at[j + 2]], bufs[b], sems[b])
        return c

    lax.fori_loop(0, RPW // 8, group, 0)
    plsc.subcore_barrier()

    @pl.when(cid == 0)
    def _():
        pltpu.sync_copy(acc_sh.at[pl.ds(base, NPT)],
                        out0_hbm.at[pl.ds(base, NPT)])

    @pl.when(cid == 1)
    def _():
        pltpu.sync_copy(acc_sh.at[pl.ds(base, NPT)],
                        out1_hbm.at[pl.ds(base, NPT)])


@functools.cache
def _sc_kernels():
    mesh = plsc.VectorSubcoreMesh(
        core_axis_name="c", subcore_axis_name="s",
        num_cores=NC, num_subcores=NS)
    sc_deg = pl.kernel(
        _sc_deg_body,
        out_type=(jax.ShapeDtypeStruct((DGR, 128), jnp.float32),
                  jax.ShapeDtypeStruct((DGR, 128), jnp.float32)),
        mesh=mesh,
        scratch_types=[
            pltpu.VMEM((EPT,), jnp.int32),
            pltpu.VMEM((TABR, 128), jnp.float32),
            pltpu.SemaphoreType.DMA,
            pltpu.VMEM_SHARED((DGR, 128), jnp.float32),
        ],
        compiler_params=pltpu.CompilerParams(needs_layout_passes=False),
    )
    sc_agg = pl.kernel(
        _sc_agg_body,
        out_type=(jax.ShapeDtypeStruct((N_PAD, D), jnp.float32),
                  jax.ShapeDtypeStruct((N_PAD, D), jnp.float32)),
        mesh=mesh,
        scratch_types=[
            pltpu.VMEM((RPW, KE), jnp.int32),
            pltpu.VMEM((8, KE), jnp.int32),
            pltpu.VMEM((KE, D), jnp.float32),
            pltpu.VMEM((KE, D), jnp.float32),
            pltpu.VMEM((ZB, D), jnp.float32),
            pltpu.SemaphoreType.DMA,
            pltpu.SemaphoreType.DMA,
            pltpu.VMEM_SHARED((N_PAD, D), jnp.float32),
        ],
    )
    return sc_deg, sc_agg


def _tc_pre_body(d0_ref, d1_ref, x_ref, w1_ref, m1_ref, dinv_ref):
    deg = jnp.sum(d0_ref[...] + d1_ref[...], axis=1, keepdims=True) + 1.0
    dinv = lax.rsqrt(deg)
    t = jnp.dot(x_ref[...], w1_ref[...], preferred_element_type=jnp.float32)
    m1_ref[...] = dinv * t
    dinv_ref[...] = jnp.broadcast_to(dinv, (RB, DK))


def _tc_mid_body(a0_ref, a1_ref, m1_ref, dinv_ref, b1_ref, m2_ref):
    dv = dinv_ref[:, 0:1]
    h = jnp.maximum(
        dv * (a0_ref[...] + a1_ref[...] + m1_ref[...]) + b1_ref[...], 0.0)
    m2_ref[...] = dv * h


def _tc_out_body(a0_ref, a1_ref, m2_ref, dinv_ref, w2_ref, b2_ref,
                 w3_ref, b3_ref, mu_ref, lv_ref):
    dv = dinv_ref[:, 0:1]
    g = dv * (a0_ref[...] + a1_ref[...] + m2_ref[...])
    mu_ref[...] = jnp.dot(
        g, w2_ref[...], preferred_element_type=jnp.float32) + b2_ref[...]
    lv_ref[...] = jnp.dot(
        g, w3_ref[...], preferred_element_type=jnp.float32) + b3_ref[...]


def _row_spec(w):
    return pl.BlockSpec((RB, w), lambda i: (i, 0))


def _full_spec(h, w):
    return pl.BlockSpec((h, w), lambda i: (0, 0))


_GRID = (N_PAD // RB,)

_tc_pre = pl.pallas_call(
    _tc_pre_body,
    grid=_GRID,
    in_specs=[_row_spec(DK), _row_spec(DK), _row_spec(D), _full_spec(D, D)],
    out_specs=[_row_spec(D), _row_spec(DK)],
    out_shape=[jax.ShapeDtypeStruct((N, D), jnp.float32),
               jax.ShapeDtypeStruct((N, DK), jnp.float32)],
)

_tc_mid = pl.pallas_call(
    _tc_mid_body,
    grid=_GRID,
    in_specs=[_row_spec(D), _row_spec(D), _row_spec(D), _row_spec(DK),
              _full_spec(1, D)],
    out_specs=_row_spec(D),
    out_shape=jax.ShapeDtypeStruct((N, D), jnp.float32),
)

_tc_out = pl.pallas_call(
    _tc_out_body,
    grid=_GRID,
    in_specs=[_row_spec(D), _row_spec(D), _row_spec(D), _row_spec(DK),
              _full_spec(D, D_OUT), _full_spec(1, D_OUT),
              _full_spec(D, D_OUT), _full_spec(1, D_OUT)],
    out_specs=[_row_spec(D_OUT), _row_spec(D_OUT)],
    out_shape=[jax.ShapeDtypeStruct((N, D_OUT), jnp.float32),
               jax.ShapeDtypeStruct((N, D_OUT), jnp.float32)],
)


def kernel(x, edge_index, W1, b1, W2, b2, W3, b3):
    sc_deg, sc_agg = _sc_kernels()
    src2 = edge_index[0].reshape(ROWS_E, KE)
    dst2 = edge_index[1].reshape(ROWS_E, KE)
    deg0, deg1 = sc_deg(edge_index[1])
    m1, dinv16 = _tc_pre(deg0.reshape(N_PAD, DK), deg1.reshape(N_PAD, DK),
                         x, W1)
    a0, a1 = sc_agg(m1, src2, dst2)
    m2 = _tc_mid(a0, a1, m1, dinv16, b1.reshape(1, D))
    b0, b1agg = sc_agg(m2, src2, dst2)
    mu, lv = _tc_out(b0, b1agg, m2, dinv16, W2, b2.reshape(1, D_OUT),
                     W3, b3.reshape(1, D_OUT))
    return mu, lv
